# Initial kernel scaffold; baseline (speedup 1.0000x reference)
import jax
import jax.numpy as jnp
from jax.experimental import pallas as pl


def kernel(embedded_sequence, weight):
    raise NotImplementedError("write your pallas kernel here")


# speedup vs baseline: 9.6773x; 9.6773x over previous
import jax
import jax.numpy as jnp
from jax.experimental import pallas as pl

_VOCAB = 156
_DIM = 19


def _cosine_top1_kernel(x_ref, w_ref, cos_ref, idx_ref):
    x = x_ref[...]
    w = w_ref[...]
    dot = jax.lax.dot_general(
        x, w, (((1,), (1,)), ((), ())), preferred_element_type=jnp.float32
    )
    norm_e = jnp.sqrt(jnp.sum(x * x, axis=1, keepdims=True))
    norm_w = jnp.sqrt(jnp.sum(w * w, axis=1, keepdims=True))
    cos = dot / jnp.maximum(norm_e * norm_w.reshape(1, _VOCAB), 1e-8)
    cos_ref[...] = cos
    idx_ref[...] = jnp.argmax(cos, axis=1, keepdims=True).astype(jnp.int32)


def kernel(embedded_sequence, weight):
    B, L, D = embedded_sequence.shape
    N = B * L
    R = 3200
    x2 = embedded_sequence.reshape(N, D)
    cos, idx = pl.pallas_call(
        _cosine_top1_kernel,
        grid=(N // R,),
        in_specs=[
            pl.BlockSpec((R, _DIM), lambda i: (i, 0)),
            pl.BlockSpec((_VOCAB, _DIM), lambda i: (0, 0)),
        ],
        out_specs=[
            pl.BlockSpec((R, _VOCAB), lambda i: (i, 0)),
            pl.BlockSpec((R, 1), lambda i: (i, 0)),
        ],
        out_shape=[
            jax.ShapeDtypeStruct((N, _VOCAB), jnp.float32),
            jax.ShapeDtypeStruct((N, 1), jnp.int32),
        ],
    )(x2, weight)
    return cos.reshape(B, L, _VOCAB), idx.reshape(B, L, 1)
